# baseline (speedup 1.0000x reference)
import functools

import jax
import jax.numpy as jnp
import numpy as np
from jax import lax
from jax.experimental import pallas as pl
from jax.experimental.pallas import tpu as pltpu
from jax.experimental.pallas import tpu_sc as plsc

N = 10000
E = 320000
D = 128
H = 64
C = 16
M = 13
NCLS = 16

MH = 7
MCK = MH * C
CK = 128
EB = 112
NTILES = 16
NSC = 2
EPT = 20160
EPAD = EPT * NTILES
NBATCH = EPT // EB
NTRI = NBATCH // 3
NACC = 10240
BN = 1000
NBLK = N // BN


def _sc_edge_body(zc_hbm, wte_hbm, sd_hbm, out_hbm,
                  acc, wbuf, sdbuf, rows_v,
                  sg0, sg1, sg2, ss0, ss1, ss2):
    c = lax.axis_index("c")
    s = lax.axis_index("s")
    cN = c * N
    sg = (sg0, sg1, sg2)
    ss = (ss0, ss1, ss2)

    def _zrow(i, _):
        for j in range(CK // C):
            rows_v[0, i, pl.ds(j * C, C)] = jnp.zeros((C,), jnp.float32)
        return 0
    lax.fori_loop(0, EB, _zrow, 0)
    rows_per_tile = NACC // NTILES
    for k in range(rows_per_tile // 80):
        pltpu.sync_copy(rows_v.at[0, pl.ds(0, 80)],
                        acc.at[pl.ds(s * rows_per_tile + k * 80, 80)])
    plsc.subcore_barrier()

    def _load_gather(j, b):
        row = (c * NTILES + s) * NBATCH + b
        pltpu.sync_copy(wte_hbm.at[row], wbuf.at[j])
        pltpu.sync_copy(sd_hbm.at[row], sdbuf.at[j])
        pltpu.async_copy(zc_hbm.at[sdbuf.at[j, 0]], rows_v.at[j], sg[j])

    def _wait_gather(j):
        pltpu.make_async_copy(zc_hbm.at[sdbuf.at[j, 0]], rows_v.at[j], sg[j]).wait()

    def _scatter(j):
        pltpu.async_copy(rows_v.at[j], acc.at[sdbuf.at[j, 1]], ss[j], add=True)

    def _wait_scatter(j):
        pltpu.make_async_copy(rows_v.at[j], acc.at[sdbuf.at[j, 1]], ss[j]).wait()

    def _compute(j):
        @plsc.parallel_loop(0, EB // 16, unroll=2)
        def _group(g):
            base = g * 16
            wch = [wbuf[j, jj, pl.ds(base, 16)] for jj in range(MH)]
            for il in range(16):
                i = base + il
                sel = jnp.full((16,), il, dtype=jnp.int32)
                for jj in range(MH):
                    wj = wch[jj].at[sel].get(mode="promise_in_bounds")
                    seg = rows_v[j, i, pl.ds(jj * C, C)]
                    rows_v[j, i, pl.ds(jj * C, C)] = seg * wj

    _load_gather(0, 0)
    _load_gather(1, 1)

    def _triple(p, _):
        for k in range(3):
            b = 3 * p + k
            j2 = (k + 2) % 3

            @pl.when(b + 2 < NBATCH)
            def _prefetch():
                @pl.when(b >= 1)
                def _drain():
                    _wait_scatter(j2)
                _load_gather(j2, b + 2)

            _wait_gather(k)
            _compute(k)
            _scatter(k)
        return 0

    lax.fori_loop(0, NTRI, _triple, 0)
    for j in range(3):
        _wait_scatter(j)
    plsc.subcore_barrier()

    for t in range(N // 400):
        @pl.when(s == (t % NTILES))
        def _copy_chunk(t=t):
            pltpu.sync_copy(acc.at[pl.ds(t * 400, 400)],
                            out_hbm.at[pl.ds(cN + t * 400, 400)])


@functools.cache
def _sc_edge_kernel():
    return pl.kernel(
        _sc_edge_body,
        out_type=jax.ShapeDtypeStruct((NSC * N, CK), jnp.float32),
        mesh=plsc.VectorSubcoreMesh(core_axis_name="c", subcore_axis_name="s",
                                    num_cores=NSC, num_subcores=NTILES),
        scratch_types=[
            pltpu.VMEM_SHARED((NACC, CK), jnp.float32),
            pltpu.VMEM((3, 8, EB), jnp.float32),
            pltpu.VMEM((3, 2, EB), jnp.int32),
            pltpu.VMEM((3, EB, CK), jnp.float32),
            pltpu.SemaphoreType.DMA,
            pltpu.SemaphoreType.DMA,
            pltpu.SemaphoreType.DMA,
            pltpu.SemaphoreType.DMA,
            pltpu.SemaphoreType.DMA,
            pltpu.SemaphoreType.DMA,
        ],
    )


def _sc_edge(zc, wte, sd):
    return _sc_edge_kernel()(zc, wte, sd)


def _tc1_body(h_ref, w1_ref, b1_ref, wc_ref, out_ref):
    z = jnp.dot(h_ref[...], w1_ref[...], preferred_element_type=jnp.float32)
    z = z + b1_ref[...]
    out_ref[...] = jnp.dot(z, wc_ref[...], preferred_element_type=jnp.float32)


def _attention(s0, s1, a0, a1, msk, r0, r1):
    sc = (jnp.dot(jnp.tanh(s0), a0, preferred_element_type=jnp.float32)
          + jnp.dot(jnp.tanh(s1), a1, preferred_element_type=jnp.float32)
          + msk)
    mx = jnp.max(sc, axis=1, keepdims=True)
    ex = jnp.exp(sc - mx)
    al = ex / jnp.sum(ex, axis=1, keepdims=True)
    x0 = jnp.maximum(s0 * jnp.dot(al, r0, preferred_element_type=jnp.float32), 0.0)
    x1 = jnp.maximum(s1 * jnp.dot(al, r1, preferred_element_type=jnp.float32), 0.0)
    return x0, x1


def _tc2_body(s0_ref, s1_ref, a0_ref, a1_ref, msk_ref, r0_ref, r1_ref,
              w2a_ref, w2b_ref, b2_ref, wc_ref, out_ref):
    x0, x1 = _attention(s0_ref[...], s1_ref[...], a0_ref[...], a1_ref[...],
                        msk_ref[...], r0_ref[...], r1_ref[...])
    z2 = (jnp.dot(x0, w2a_ref[...], preferred_element_type=jnp.float32)
          + jnp.dot(x1, w2b_ref[...], preferred_element_type=jnp.float32)
          + b2_ref[...])
    out_ref[...] = jnp.dot(z2, wc_ref[...], preferred_element_type=jnp.float32)


def _tc3_body(s0_ref, s1_ref, a0_ref, a1_ref, msk_ref, r0_ref, r1_ref,
              wda_ref, wdb_ref, bd_ref, out_ref):
    x0, x1 = _attention(s0_ref[...], s1_ref[...], a0_ref[...], a1_ref[...],
                        msk_ref[...], r0_ref[...], r1_ref[...])
    lg = (jnp.dot(x0, wda_ref[...], preferred_element_type=jnp.float32)
          + jnp.dot(x1, wdb_ref[...], preferred_element_type=jnp.float32)
          + bd_ref[...])
    mx = jnp.max(lg, axis=1, keepdims=True)
    lse = jnp.log(jnp.sum(jnp.exp(lg - mx), axis=1, keepdims=True))
    out_ref[...] = lg - mx - lse


def _full(shape):
    return pl.BlockSpec(shape, lambda *_: tuple(0 for _ in shape))


def _np_consts():
    one0 = np.zeros((MH, NCLS), np.float32)
    one1 = np.zeros((MH, NCLS), np.float32)
    for j in range(MH):
        one0[j, j] = 1.0
        if MH + j < M + 1:
            one1[j, MH + j] = 1.0
    r0 = np.zeros((NCLS, CK), np.float32)
    r1 = np.zeros((NCLS, CK), np.float32)
    for j in range(MH):
        r0[j, j * C:(j + 1) * C] = 1.0
        r1[MH + j, j * C:(j + 1) * C] = 1.0
    msk = np.zeros((1, NCLS), np.float32)
    msk[0, M:] = -1e30
    return one0, one1, r0, r1, msk


_ONE0, _ONE1, _R0, _R1, _MSK = _np_consts()


def _pad_rows(x):
    return jnp.concatenate(
        [x, jnp.zeros((CK - x.shape[0],) + x.shape[1:], jnp.float32)], axis=0)


def _att_maps(att):
    att_p = jnp.concatenate([att, jnp.zeros((1, C), jnp.float32)], axis=0)
    a0 = (att_p[:MH, :, None] * jnp.asarray(_ONE0)[:, None, :]).reshape(MCK, NCLS)
    a1 = (att_p[MH:, :, None] * jnp.asarray(_ONE1)[:, None, :]).reshape(MCK, NCLS)
    return _pad_rows(a0), _pad_rows(a1)


def _wc_stack(wc):
    wc_p = jnp.concatenate([wc, jnp.zeros((1, H, C), jnp.float32)], axis=0)
    zpad = jnp.zeros((H, CK - MCK), jnp.float32)
    h0 = jnp.concatenate([jnp.transpose(wc_p[:MH], (1, 0, 2)).reshape(H, MCK), zpad], axis=1)
    h1 = jnp.concatenate([jnp.transpose(wc_p[MH:], (1, 0, 2)).reshape(H, MCK), zpad], axis=1)
    return jnp.concatenate([h0, h1], axis=0)


def kernel(h, edge_index, motif_weights, W1, b1, Wc1, att1, W2, b2, Wc2, att2,
           Wd, bd):
    src = jnp.pad(edge_index[0], (0, EPAD - E))
    dst = jnp.pad(edge_index[1], (0, EPAD - E), constant_values=N)
    mw = jnp.pad(motif_weights, ((0, 1), (0, EPAD - E)))
    wte = jnp.pad(mw.reshape(NSC, MH, EPAD), ((0, 0), (0, 1), (0, 0)))
    sd = jnp.stack([jnp.stack([src, dst]), jnp.stack([src + N, dst])])
    nrow = NTILES * NBATCH
    wteb = wte.reshape(NSC, 8, nrow, EB).transpose(0, 2, 1, 3).reshape(
        NSC * nrow, 8, EB)
    sdb = sd.reshape(NSC, 2, nrow, EB).transpose(0, 2, 1, 3).reshape(
        NSC * nrow, 2, EB)

    wcs1 = _wc_stack(Wc1)
    wcs2 = _wc_stack(Wc2)
    a10, a11 = _att_maps(att1)
    a20, a21 = _att_maps(att2)
    r0 = jnp.asarray(_R0)
    r1 = jnp.asarray(_R1)
    msk = jnp.asarray(_MSK)
    b1r = b1.reshape(1, H)
    b2r = b2.reshape(1, H)
    bdr = bd.reshape(1, NCLS)
    w2a = _pad_rows(W2[:MCK])
    w2b = _pad_rows(W2[MCK:])
    wda = _pad_rows(Wd[:MCK])
    wdb = _pad_rows(Wd[MCK:])

    zc1 = pl.pallas_call(
        _tc1_body,
        grid=(NSC, NBLK),
        in_specs=[
            pl.BlockSpec((BN, D), lambda c, i: (i, 0)),
            pl.BlockSpec((D, H), lambda c, i: (0, 0)),
            pl.BlockSpec((1, H), lambda c, i: (0, 0)),
            pl.BlockSpec((H, CK), lambda c, i: (c, 0)),
        ],
        out_specs=pl.BlockSpec((BN, CK), lambda c, i: (c * NBLK + i, 0)),
        out_shape=jax.ShapeDtypeStruct((NSC * N, CK), jnp.float32),
    )(h, W1, b1r, wcs1)

    stacked1 = _sc_edge(zc1, wteb, sdb)

    half0 = pl.BlockSpec((BN, CK), lambda c, i: (i, 0))
    half1 = pl.BlockSpec((BN, CK), lambda c, i: (NBLK + i, 0))
    zc2 = pl.pallas_call(
        _tc2_body,
        grid=(NSC, NBLK),
        in_specs=[
            half0, half1,
            _full((CK, NCLS)), _full((CK, NCLS)), _full((1, NCLS)),
            _full((NCLS, CK)), _full((NCLS, CK)),
            _full((CK, H)), _full((CK, H)), _full((1, H)),
            pl.BlockSpec((H, CK), lambda c, i: (c, 0)),
        ],
        out_specs=pl.BlockSpec((BN, CK), lambda c, i: (c * NBLK + i, 0)),
        out_shape=jax.ShapeDtypeStruct((NSC * N, CK), jnp.float32),
    )(stacked1, stacked1, a10, a11, msk, r0, r1, w2a, w2b, b2r, wcs2)

    stacked2 = _sc_edge(zc2, wteb, sdb)

    half0b = pl.BlockSpec((BN, CK), lambda i: (i, 0))
    half1b = pl.BlockSpec((BN, CK), lambda i: (NBLK + i, 0))
    out = pl.pallas_call(
        _tc3_body,
        grid=(NBLK,),
        in_specs=[
            half0b, half1b,
            _full((CK, NCLS)), _full((CK, NCLS)), _full((1, NCLS)),
            _full((NCLS, CK)), _full((NCLS, CK)),
            _full((CK, NCLS)), _full((CK, NCLS)), _full((1, NCLS)),
        ],
        out_specs=pl.BlockSpec((BN, NCLS), lambda i: (i, 0)),
        out_shape=jax.ShapeDtypeStruct((N, NCLS), jnp.float32),
    )(stacked2, stacked2, a20, a21, msk, r0, r1, wda, wdb, bdr)
    return out


# speedup vs baseline: 1.2700x; 1.2700x over previous
import functools

import jax
import jax.numpy as jnp
import numpy as np
from jax import lax
from jax.experimental import pallas as pl
from jax.experimental.pallas import tpu as pltpu
from jax.experimental.pallas import tpu_sc as plsc

N = 10000
E = 320000
D = 128
H = 64
C = 16
M = 13
NCLS = 16

MH = 7
MCK = MH * C
CK = 128
EB = 112
NTILES = 16
NSC = 2
EPT = 20160
EPAD = EPT * NTILES
NBATCH = EPT // EB
NTRI = NBATCH // 3
NACC = 10240
BN = 1000
NBLK = N // BN


def _sc_edge_body(zc_hbm, wte_hbm, sd_hbm, out_hbm,
                  acc, wbuf, sdbuf, rows_v,
                  sg0, sg1, sg2, ss0, ss1, ss2, sp0, sp1, sp2):
    c = lax.axis_index("c")
    s = lax.axis_index("s")
    cN = c * N
    sg = (sg0, sg1, sg2)
    ss = (ss0, ss1, ss2)
    sp = (sp0, sp1, sp2)

    def _zrow(i, _):
        for j in range(CK // C):
            rows_v[0, i, pl.ds(j * C, C)] = jnp.zeros((C,), jnp.float32)
        return 0
    lax.fori_loop(0, EB, _zrow, 0)
    rows_per_tile = NACC // NTILES
    for k in range(rows_per_tile // 80):
        pltpu.sync_copy(rows_v.at[0, pl.ds(0, 80)],
                        acc.at[pl.ds(s * rows_per_tile + k * 80, 80)])
    plsc.subcore_barrier()

    def _load_payload(j, b):
        row = (c * NTILES + s) * NBATCH + b
        pltpu.async_copy(wte_hbm.at[row], wbuf.at[j], sp[j])
        pltpu.async_copy(sd_hbm.at[row], sdbuf.at[j], sp[j])

    def _wait_payload(j, b):
        row = (c * NTILES + s) * NBATCH + b
        pltpu.make_async_copy(wte_hbm.at[row], wbuf.at[j], sp[j]).wait()
        pltpu.make_async_copy(sd_hbm.at[row], sdbuf.at[j], sp[j]).wait()

    def _gather(j):
        pltpu.async_copy(zc_hbm.at[sdbuf.at[j, 0]], rows_v.at[j], sg[j])

    def _wait_gather(j):
        pltpu.make_async_copy(zc_hbm.at[sdbuf.at[j, 0]], rows_v.at[j], sg[j]).wait()

    def _scatter(j):
        pltpu.async_copy(rows_v.at[j], acc.at[sdbuf.at[j, 1]], ss[j], add=True)

    def _wait_scatter(j):
        pltpu.make_async_copy(rows_v.at[j], acc.at[sdbuf.at[j, 1]], ss[j]).wait()

    def _compute(j):
        @plsc.parallel_loop(0, EB // 16, unroll=2)
        def _group(g):
            base = g * 16
            wch = [wbuf[j, jj, pl.ds(base, 16)] for jj in range(MH)]
            for il in range(16):
                i = base + il
                sel = jnp.full((16,), il, dtype=jnp.int32)
                for jj in range(MH):
                    wj = wch[jj].at[sel].get(mode="promise_in_bounds")
                    seg = rows_v[j, i, pl.ds(jj * C, C)]
                    rows_v[j, i, pl.ds(jj * C, C)] = seg * wj

    _load_payload(0, 0)
    _wait_payload(0, 0)
    _gather(0)
    _load_payload(1, 1)

    def _triple(p, _):
        for k in range(3):
            b = 3 * p + k
            j1 = (k + 1) % 3
            j2 = (k + 2) % 3

            @pl.when(b + 2 < NBATCH)
            def _prefetch():
                @pl.when(b >= 1)
                def _drain():
                    _wait_scatter(j2)
                _load_payload(j2, b + 2)

            @pl.when(b + 1 < NBATCH)
            def _start_gather():
                _wait_payload(j1, b + 1)
                _gather(j1)

            _wait_gather(k)
            _compute(k)
            _scatter(k)
        return 0

    lax.fori_loop(0, NTRI, _triple, 0)
    for j in range(3):
        _wait_scatter(j)
    plsc.subcore_barrier()

    for t in range(N // 400):
        @pl.when(s == (t % NTILES))
        def _copy_chunk(t=t):
            pltpu.sync_copy(acc.at[pl.ds(t * 400, 400)],
                            out_hbm.at[pl.ds(cN + t * 400, 400)])


@functools.cache
def _sc_edge_kernel():
    return pl.kernel(
        _sc_edge_body,
        out_type=jax.ShapeDtypeStruct((NSC * N, CK), jnp.float32),
        mesh=plsc.VectorSubcoreMesh(core_axis_name="c", subcore_axis_name="s",
                                    num_cores=NSC, num_subcores=NTILES),
        scratch_types=[
            pltpu.VMEM_SHARED((NACC, CK), jnp.float32),
            pltpu.VMEM((3, 8, EB), jnp.float32),
            pltpu.VMEM((3, 2, EB), jnp.int32),
            pltpu.VMEM((3, EB, CK), jnp.float32),
            pltpu.SemaphoreType.DMA,
            pltpu.SemaphoreType.DMA,
            pltpu.SemaphoreType.DMA,
            pltpu.SemaphoreType.DMA,
            pltpu.SemaphoreType.DMA,
            pltpu.SemaphoreType.DMA,
            pltpu.SemaphoreType.DMA,
            pltpu.SemaphoreType.DMA,
            pltpu.SemaphoreType.DMA,
        ],
    )


def _sc_edge(zc, wte, sd):
    return _sc_edge_kernel()(zc, wte, sd)


def _tc1_body(h_ref, w1_ref, b1_ref, wc_ref, out_ref):
    z = jnp.dot(h_ref[...], w1_ref[...], preferred_element_type=jnp.float32)
    z = z + b1_ref[...]
    out_ref[...] = jnp.dot(z, wc_ref[...], preferred_element_type=jnp.float32)


def _attention(s0, s1, a0, a1, msk, r0, r1):
    sc = (jnp.dot(jnp.tanh(s0), a0, preferred_element_type=jnp.float32)
          + jnp.dot(jnp.tanh(s1), a1, preferred_element_type=jnp.float32)
          + msk)
    mx = jnp.max(sc, axis=1, keepdims=True)
    ex = jnp.exp(sc - mx)
    al = ex / jnp.sum(ex, axis=1, keepdims=True)
    x0 = jnp.maximum(s0 * jnp.dot(al, r0, preferred_element_type=jnp.float32), 0.0)
    x1 = jnp.maximum(s1 * jnp.dot(al, r1, preferred_element_type=jnp.float32), 0.0)
    return x0, x1


def _tc2_body(s0_ref, s1_ref, a0_ref, a1_ref, msk_ref, r0_ref, r1_ref,
              w2a_ref, w2b_ref, b2_ref, wc_ref, out_ref):
    x0, x1 = _attention(s0_ref[...], s1_ref[...], a0_ref[...], a1_ref[...],
                        msk_ref[...], r0_ref[...], r1_ref[...])
    z2 = (jnp.dot(x0, w2a_ref[...], preferred_element_type=jnp.float32)
          + jnp.dot(x1, w2b_ref[...], preferred_element_type=jnp.float32)
          + b2_ref[...])
    out_ref[...] = jnp.dot(z2, wc_ref[...], preferred_element_type=jnp.float32)


def _tc3_body(s0_ref, s1_ref, a0_ref, a1_ref, msk_ref, r0_ref, r1_ref,
              wda_ref, wdb_ref, bd_ref, out_ref):
    x0, x1 = _attention(s0_ref[...], s1_ref[...], a0_ref[...], a1_ref[...],
                        msk_ref[...], r0_ref[...], r1_ref[...])
    lg = (jnp.dot(x0, wda_ref[...], preferred_element_type=jnp.float32)
          + jnp.dot(x1, wdb_ref[...], preferred_element_type=jnp.float32)
          + bd_ref[...])
    mx = jnp.max(lg, axis=1, keepdims=True)
    lse = jnp.log(jnp.sum(jnp.exp(lg - mx), axis=1, keepdims=True))
    out_ref[...] = lg - mx - lse


def _full(shape):
    return pl.BlockSpec(shape, lambda *_: tuple(0 for _ in shape))


def _np_consts():
    one0 = np.zeros((MH, NCLS), np.float32)
    one1 = np.zeros((MH, NCLS), np.float32)
    for j in range(MH):
        one0[j, j] = 1.0
        if MH + j < M + 1:
            one1[j, MH + j] = 1.0
    r0 = np.zeros((NCLS, CK), np.float32)
    r1 = np.zeros((NCLS, CK), np.float32)
    for j in range(MH):
        r0[j, j * C:(j + 1) * C] = 1.0
        r1[MH + j, j * C:(j + 1) * C] = 1.0
    msk = np.zeros((1, NCLS), np.float32)
    msk[0, M:] = -1e30
    return one0, one1, r0, r1, msk


_ONE0, _ONE1, _R0, _R1, _MSK = _np_consts()


def _pad_rows(x):
    return jnp.concatenate(
        [x, jnp.zeros((CK - x.shape[0],) + x.shape[1:], jnp.float32)], axis=0)


def _att_maps(att):
    att_p = jnp.concatenate([att, jnp.zeros((1, C), jnp.float32)], axis=0)
    a0 = (att_p[:MH, :, None] * jnp.asarray(_ONE0)[:, None, :]).reshape(MCK, NCLS)
    a1 = (att_p[MH:, :, None] * jnp.asarray(_ONE1)[:, None, :]).reshape(MCK, NCLS)
    return _pad_rows(a0), _pad_rows(a1)


def _wc_stack(wc):
    wc_p = jnp.concatenate([wc, jnp.zeros((1, H, C), jnp.float32)], axis=0)
    zpad = jnp.zeros((H, CK - MCK), jnp.float32)
    h0 = jnp.concatenate([jnp.transpose(wc_p[:MH], (1, 0, 2)).reshape(H, MCK), zpad], axis=1)
    h1 = jnp.concatenate([jnp.transpose(wc_p[MH:], (1, 0, 2)).reshape(H, MCK), zpad], axis=1)
    return jnp.concatenate([h0, h1], axis=0)


def kernel(h, edge_index, motif_weights, W1, b1, Wc1, att1, W2, b2, Wc2, att2,
           Wd, bd):
    src = jnp.pad(edge_index[0], (0, EPAD - E))
    dst = jnp.pad(edge_index[1], (0, EPAD - E), constant_values=N)
    mw = jnp.pad(motif_weights, ((0, 1), (0, EPAD - E)))
    wte = jnp.pad(mw.reshape(NSC, MH, EPAD), ((0, 0), (0, 1), (0, 0)))
    sd = jnp.stack([jnp.stack([src, dst]), jnp.stack([src + N, dst])])
    nrow = NTILES * NBATCH
    wteb = wte.reshape(NSC, 8, nrow, EB).transpose(0, 2, 1, 3).reshape(
        NSC * nrow, 8, EB)
    sdb = sd.reshape(NSC, 2, nrow, EB).transpose(0, 2, 1, 3).reshape(
        NSC * nrow, 2, EB)

    wcs1 = _wc_stack(Wc1)
    wcs2 = _wc_stack(Wc2)
    a10, a11 = _att_maps(att1)
    a20, a21 = _att_maps(att2)
    r0 = jnp.asarray(_R0)
    r1 = jnp.asarray(_R1)
    msk = jnp.asarray(_MSK)
    b1r = b1.reshape(1, H)
    b2r = b2.reshape(1, H)
    bdr = bd.reshape(1, NCLS)
    w2a = _pad_rows(W2[:MCK])
    w2b = _pad_rows(W2[MCK:])
    wda = _pad_rows(Wd[:MCK])
    wdb = _pad_rows(Wd[MCK:])

    zc1 = pl.pallas_call(
        _tc1_body,
        grid=(NSC, NBLK),
        in_specs=[
            pl.BlockSpec((BN, D), lambda c, i: (i, 0)),
            pl.BlockSpec((D, H), lambda c, i: (0, 0)),
            pl.BlockSpec((1, H), lambda c, i: (0, 0)),
            pl.BlockSpec((H, CK), lambda c, i: (c, 0)),
        ],
        out_specs=pl.BlockSpec((BN, CK), lambda c, i: (c * NBLK + i, 0)),
        out_shape=jax.ShapeDtypeStruct((NSC * N, CK), jnp.float32),
    )(h, W1, b1r, wcs1)

    stacked1 = _sc_edge(zc1, wteb, sdb)

    half0 = pl.BlockSpec((BN, CK), lambda c, i: (i, 0))
    half1 = pl.BlockSpec((BN, CK), lambda c, i: (NBLK + i, 0))
    zc2 = pl.pallas_call(
        _tc2_body,
        grid=(NSC, NBLK),
        in_specs=[
            half0, half1,
            _full((CK, NCLS)), _full((CK, NCLS)), _full((1, NCLS)),
            _full((NCLS, CK)), _full((NCLS, CK)),
            _full((CK, H)), _full((CK, H)), _full((1, H)),
            pl.BlockSpec((H, CK), lambda c, i: (c, 0)),
        ],
        out_specs=pl.BlockSpec((BN, CK), lambda c, i: (c * NBLK + i, 0)),
        out_shape=jax.ShapeDtypeStruct((NSC * N, CK), jnp.float32),
    )(stacked1, stacked1, a10, a11, msk, r0, r1, w2a, w2b, b2r, wcs2)

    stacked2 = _sc_edge(zc2, wteb, sdb)

    half0b = pl.BlockSpec((BN, CK), lambda i: (i, 0))
    half1b = pl.BlockSpec((BN, CK), lambda i: (NBLK + i, 0))
    out = pl.pallas_call(
        _tc3_body,
        grid=(NBLK,),
        in_specs=[
            half0b, half1b,
            _full((CK, NCLS)), _full((CK, NCLS)), _full((1, NCLS)),
            _full((NCLS, CK)), _full((NCLS, CK)),
            _full((CK, NCLS)), _full((CK, NCLS)), _full((1, NCLS)),
        ],
        out_specs=pl.BlockSpec((BN, NCLS), lambda i: (i, 0)),
        out_shape=jax.ShapeDtypeStruct((N, NCLS), jnp.float32),
    )(stacked2, stacked2, a20, a21, msk, r0, r1, wda, wdb, bdr)
    return out
